# baseline (speedup 1.0000x reference)
import jax
import jax.numpy as jnp
from jax import lax
from jax.experimental import pallas as pl
from jax.experimental.pallas import tpu as pltpu
from jax.experimental.pallas import tpu_sc as plsc

_F = 26
_K = 16
_B = 4096
_NC = 2
_NS = 16
_NW = _NC * _NS
_BPW = _B // _NW
_G = _BPW // _K


def _fm_body(x_ref, emb_ref, w1_ref, w0_ref, fm_ref, prob_ref,
             idx_v, rows_v, w1v_v, w0_v, rbuf_v, out_v, sem, wsem):
    wid = lax.axis_index("s") * _NC + lax.axis_index("c")
    base = wid * _BPW

    pltpu.sync_copy(x_ref.at[:, pl.ds(base, _BPW)], idx_v)
    pltpu.sync_copy(w0_ref, w0_v)

    emb_copies = [
        pltpu.async_copy(emb_ref.at[idx_v.at[f]], rows_v.at[f], sem)
        for f in range(_F)
    ]
    w1_copies = [
        pltpu.async_copy(w1_ref.at[idx_v.at[f]], w1v_v.at[f], wsem)
        for f in range(_F)
    ]
    for c in emb_copies:
        c.wait()
    for c in w1_copies:
        c.wait()

    w0s = w0_v[...]
    lane = lax.iota(jnp.int32, _K)

    def gbody(g, carry):
        for jj in range(_K):
            j = g * _K + jj
            s = rows_v[0, j]
            q = s * s
            for f in range(1, _F):
                v = rows_v[f, j]
                s = s + v
                q = q + v * v
            r = s * s - q
            plsc.store_scatter(rbuf_v, [lane * _K + jj], r)
        fm2 = rbuf_v[pl.ds(0, _K)]
        for k in range(1, _K):
            fm2 = fm2 + rbuf_v[pl.ds(k * _K, _K)]
        sl = pl.ds(g * _K, _K)
        w1s = w1v_v[0, sl]
        for f in range(1, _F):
            w1s = w1s + w1v_v[f, sl]
        fm = w0s + w1s + 0.5 * fm2
        out_v[0, sl] = fm
        out_v[1, sl] = 1.0 / (1.0 + jnp.exp(-fm))
        return carry

    lax.fori_loop(0, _G, gbody, 0)

    pltpu.sync_copy(out_v.at[0], fm_ref.at[pl.ds(base, _BPW)])
    pltpu.sync_copy(out_v.at[1], prob_ref.at[pl.ds(base, _BPW)])


def kernel(x, emb_v, w1, w0):
    x32 = x.astype(jnp.int32)
    w1f = w1.reshape(-1)
    w0v = jnp.broadcast_to(w0.astype(jnp.float32).reshape(1), (_K,))
    mesh = plsc.VectorSubcoreMesh(core_axis_name="c", subcore_axis_name="s")
    fm_flat, prob_flat = pl.kernel(
        _fm_body,
        out_type=(
            jax.ShapeDtypeStruct((_B,), jnp.float32),
            jax.ShapeDtypeStruct((_B,), jnp.float32),
        ),
        mesh=mesh,
        compiler_params=pltpu.CompilerParams(
            needs_layout_passes=False, use_tc_tiling_on_sc=False),
        scratch_types=[
            pltpu.VMEM((_F, _BPW), jnp.int32),
            pltpu.VMEM((_F, _BPW, _K), jnp.float32),
            pltpu.VMEM((_F, _BPW), jnp.float32),
            pltpu.VMEM((_K,), jnp.float32),
            pltpu.VMEM((_K * _K,), jnp.float32),
            pltpu.VMEM((2, _BPW), jnp.float32),
            pltpu.SemaphoreType.DMA,
            pltpu.SemaphoreType.DMA,
        ],
    )(x32, emb_v, w1f, w0v)
    return fm_flat.reshape(_B, 1), prob_flat.reshape(_B, 1)
